# Initial kernel scaffold; baseline (speedup 1.0000x reference)
import jax
import jax.numpy as jnp
from jax.experimental import pallas as pl


def kernel(x, tables):
    raise NotImplementedError("write your pallas kernel here")


# speedup vs baseline: 9.8966x; 9.8966x over previous
import functools

import jax
import jax.numpy as jnp
import numpy as np
from jax import lax
from jax.experimental import pallas as pl
from jax.experimental.pallas import tpu as pltpu
from jax.experimental.pallas import tpu_sc as plsc

LEVELS = 16
FEATURES = 2
HASHMAP_SIZE = 524288
MASK = np.int32(HASHMAP_SIZE - 1)
BASE_RES = 16
PER_LEVEL_SCALE = 1.5
N_POINTS = 1048576

P1 = np.int32(73856093)
P2 = np.int32(19349663)
P3 = np.int32(83492791)

RES = [int(BASE_RES * PER_LEVEL_SCALE**l) for l in range(LEVELS)]

NUM_CORES = 2
NUM_SUBCORES = 16
NUM_WORKERS = NUM_CORES * NUM_SUBCORES
PTS_PER_WORKER = N_POINTS // NUM_WORKERS
C = 512
CHUNKS = PTS_PER_WORKER // C


def _sc_encode(x_flat, tab_flat):
    mesh = plsc.VectorSubcoreMesh(core_axis_name="c", subcore_axis_name="s")

    @functools.partial(
        pl.kernel,
        mesh=mesh,
        out_type=jax.ShapeDtypeStruct((N_POINTS * LEVELS, FEATURES), jnp.float32),
        compiler_params=pltpu.CompilerParams(
            needs_layout_passes=False, use_tc_tiling_on_sc=False
        ),
        scratch_types=[
            pltpu.VMEM((3 * C,), jnp.float32),
            pltpu.VMEM((LEVELS * C,), jnp.int32),
            pltpu.VMEM((LEVELS * C, FEATURES), jnp.float32),
            pltpu.SemaphoreType.DMA,
        ],
    )
    def k(x_hbm, tab_hbm, out_hbm, xbuf, idxbuf, gbuf, sem):
        wid = lax.axis_index("s") * NUM_CORES + lax.axis_index("c")
        iota = lax.iota(jnp.int32, 16)

        def chunk_body(g, carry):
            base = (wid * CHUNKS + g) * C
            pltpu.sync_copy(x_hbm.at[pl.ds(base * 3, 3 * C)], xbuf)

            def vreg_body(i, carry2):
                off0 = i * 48 + iota * 3
                cx = plsc.load_gather(xbuf, [off0])
                cy = plsc.load_gather(xbuf, [off0 + 1])
                cz = plsc.load_gather(xbuf, [off0 + 2])
                pbase = (i * 16 + iota) * LEVELS
                for l in range(LEVELS):
                    res = np.float32(RES[l])
                    xi = (cx * res).astype(jnp.int32)
                    yi = (cy * res).astype(jnp.int32)
                    zi = (cz * res).astype(jnp.int32)
                    h = (xi * P1) ^ (yi * P2) ^ (zi * P3)
                    idx = (h & MASK) + np.int32(l * HASHMAP_SIZE)
                    plsc.store_scatter(idxbuf, [pbase + np.int32(l)], idx)
                return carry2

            lax.fori_loop(0, C // 16, vreg_body, 0)
            pltpu.async_copy(tab_hbm.at[idxbuf], gbuf, sem).wait()
            pltpu.sync_copy(gbuf, out_hbm.at[pl.ds(base * LEVELS, LEVELS * C)])
            return carry

        lax.fori_loop(0, CHUNKS, chunk_body, 0)

    return k(x_flat, tab_flat)


def kernel(x, tables):
    x_flat = x.reshape(-1)
    tab_flat = tables.reshape(LEVELS * HASHMAP_SIZE, FEATURES)
    out2 = _sc_encode(x_flat, tab_flat)
    return out2.reshape(N_POINTS, LEVELS * FEATURES)


# speedup vs baseline: 69.2617x; 6.9985x over previous
import functools

import jax
import jax.numpy as jnp
import numpy as np
from jax import lax
from jax.experimental import pallas as pl
from jax.experimental.pallas import tpu as pltpu
from jax.experimental.pallas import tpu_sc as plsc

LEVELS = 16
FEATURES = 2
HASHMAP_SIZE = 524288
MASK = np.int32(HASHMAP_SIZE - 1)
BASE_RES = 16
PER_LEVEL_SCALE = 1.5
N_POINTS = 1048576

P1 = np.int32(73856093)
P2 = np.int32(19349663)
P3 = np.int32(83492791)

RES = [int(BASE_RES * PER_LEVEL_SCALE**l) for l in range(LEVELS)]

NUM_CORES = 2
NUM_SUBCORES = 16
NUM_WORKERS = NUM_CORES * NUM_SUBCORES
PTS_PER_WORKER = N_POINTS // NUM_WORKERS
C = 512
CHUNKS = PTS_PER_WORKER // C


def _sc_encode(x_flat, tab_flat):
    mesh = plsc.VectorSubcoreMesh(core_axis_name="c", subcore_axis_name="s")

    @functools.partial(
        pl.kernel,
        mesh=mesh,
        out_type=jax.ShapeDtypeStruct((N_POINTS * LEVELS * FEATURES,), jnp.float32),
        compiler_params=pltpu.CompilerParams(
            needs_layout_passes=False, use_tc_tiling_on_sc=False
        ),
        scratch_types=[
            pltpu.VMEM((3 * C,), jnp.float32),
            pltpu.VMEM((2 * LEVELS * C,), jnp.int32),
            pltpu.VMEM((2 * LEVELS * C,), jnp.float32),
            pltpu.SemaphoreType.DMA,
        ],
    )
    def k(x_hbm, tab_hbm, out_hbm, xbuf, idxbuf, gbuf, sem):
        wid = lax.axis_index("s") * NUM_CORES + lax.axis_index("c")
        iota = lax.iota(jnp.int32, 16)

        def chunk_body(g, carry):
            base = (wid * CHUNKS + g) * C
            pltpu.sync_copy(x_hbm.at[pl.ds(base * 3, 3 * C)], xbuf)

            def vreg_body(i, carry2):
                off0 = i * 48 + iota * 3
                cx = plsc.load_gather(xbuf, [off0])
                cy = plsc.load_gather(xbuf, [off0 + 1])
                cz = plsc.load_gather(xbuf, [off0 + 2])
                pbase = (i * 16 + iota) * (LEVELS * FEATURES)
                for l in range(LEVELS):
                    res = np.float32(RES[l])
                    xi = (cx * res).astype(jnp.int32)
                    yi = (cy * res).astype(jnp.int32)
                    zi = (cz * res).astype(jnp.int32)
                    h = (xi * P1) ^ (yi * P2) ^ (zi * P3)
                    v = h & MASK
                    g0 = (
                        ((v >> 7) << 8)
                        + (v & np.int32(127))
                        + np.int32(l * 1048576)
                    )
                    plsc.store_scatter(idxbuf, [pbase + np.int32(2 * l)], g0)
                    plsc.store_scatter(
                        idxbuf, [pbase + np.int32(2 * l + 1)], g0 + np.int32(128)
                    )
                return carry2

            lax.fori_loop(0, C // 16, vreg_body, 0)
            pltpu.async_copy(tab_hbm.at[idxbuf], gbuf, sem).wait()
            pltpu.sync_copy(
                gbuf,
                out_hbm.at[pl.ds(base * LEVELS * FEATURES, LEVELS * FEATURES * C)],
            )
            return carry

        lax.fori_loop(0, CHUNKS, chunk_body, 0)

    return k(x_flat, tab_flat)


def kernel(x, tables):
    x_flat = x.reshape(-1)
    tab_flat = (
        tables.reshape(LEVELS, 4096, 128, FEATURES)
        .transpose(0, 1, 3, 2)
        .reshape(-1)
    )
    out = _sc_encode(x_flat, tab_flat)
    return out.reshape(N_POINTS, LEVELS * FEATURES)


# speedup vs baseline: 142.3667x; 2.0555x over previous
import functools

import jax
import jax.numpy as jnp
import numpy as np
from jax import lax
from jax.experimental import pallas as pl
from jax.experimental.pallas import tpu as pltpu
from jax.experimental.pallas import tpu_sc as plsc

LEVELS = 16
FEATURES = 2
HASHMAP_SIZE = 524288
MASK = np.int32(HASHMAP_SIZE - 1)
BASE_RES = 16
PER_LEVEL_SCALE = 1.5
N_POINTS = 1048576

P1 = np.int32(73856093)
P2 = np.int32(19349663)
P3 = np.int32(83492791)

RES = [int(BASE_RES * PER_LEVEL_SCALE**l) for l in range(LEVELS)]

NUM_CORES = 2
NUM_SUBCORES = 16
NUM_WORKERS = NUM_CORES * NUM_SUBCORES
PTS_PER_WORKER = N_POINTS // NUM_WORKERS
C = 512
CHUNKS = PTS_PER_WORKER // C
OUT_C = LEVELS * FEATURES * C


def _sc_encode(x_flat, tab_flat):
    mesh = plsc.VectorSubcoreMesh(core_axis_name="c", subcore_axis_name="s")

    @functools.partial(
        pl.kernel,
        mesh=mesh,
        out_type=jax.ShapeDtypeStruct((N_POINTS * LEVELS * FEATURES,), jnp.float32),
        compiler_params=pltpu.CompilerParams(
            needs_layout_passes=False, use_tc_tiling_on_sc=False
        ),
        scratch_types=[
            pltpu.VMEM((2, 3 * C), jnp.float32),
            pltpu.VMEM((2, OUT_C), jnp.int32),
            pltpu.VMEM((2, OUT_C), jnp.float32),
            pltpu.SemaphoreType.DMA,
            pltpu.SemaphoreType.DMA,
            pltpu.SemaphoreType.DMA,
        ],
    )
    def k(x_hbm, tab_hbm, out_hbm, xbuf, idxbuf, gbuf, semx, semA, semB):
        wid = lax.axis_index("s") * NUM_CORES + lax.axis_index("c")
        iota = lax.iota(jnp.int32, 16)
        first = wid * CHUNKS

        def load_x(g, b):
            base = (first + g) * C
            for j in range(3):
                pltpu.async_copy(
                    x_hbm.at[pl.ds(j * N_POINTS + base, C)],
                    xbuf.at[b, pl.ds(j * C, C)],
                    semx,
                )
            pltpu.make_async_copy(
                x_hbm.at[pl.ds(0, 3 * C)], xbuf.at[b], semx
            ).wait()

        def compute_idx(b):
            def vreg_body(i, carry2):
                lanes = i * 16 + iota
                cx = xbuf[b, pl.ds(i * 16, 16)]
                cy = xbuf[b, pl.ds(C + i * 16, 16)]
                cz = xbuf[b, pl.ds(2 * C + i * 16, 16)]
                pbase = lanes * (LEVELS * FEATURES)
                for l in range(LEVELS):
                    res = np.float32(RES[l])
                    xi = (cx * res).astype(jnp.int32)
                    yi = (cy * res).astype(jnp.int32)
                    zi = (cz * res).astype(jnp.int32)
                    h = (xi * P1) ^ (yi * P2) ^ (zi * P3)
                    v = h & MASK
                    g0 = (
                        ((v >> 7) << 8)
                        + (v & np.int32(127))
                        + np.int32(l * 1048576)
                    )
                    plsc.store_scatter(
                        idxbuf.at[b], [pbase + np.int32(2 * l)], g0
                    )
                    plsc.store_scatter(
                        idxbuf.at[b],
                        [pbase + np.int32(2 * l + 1)],
                        g0 + np.int32(128),
                    )
                return carry2

            lax.fori_loop(0, C // 16, vreg_body, 0)

        def fire_gather(b, sem):
            pltpu.async_copy(tab_hbm.at[idxbuf.at[b]], gbuf.at[b], sem)

        def drain_store(g, b, sem):
            pltpu.make_async_copy(
                tab_hbm.at[idxbuf.at[b]], gbuf.at[b], sem
            ).wait()
            pltpu.sync_copy(gbuf.at[b], out_hbm.at[pl.ds((first + g) * 32, OUT_C)])

        load_x(0, 0)
        compute_idx(0)
        fire_gather(0, semA)
        load_x(1, 1)
        compute_idx(1)
        fire_gather(1, semB)

        def pipe_body(jj, carry):
            g = jj * 2
            drain_store(g, 0, semA)
            load_x(g + 2, 0)
            compute_idx(0)
            fire_gather(0, semA)
            drain_store(g + 1, 1, semB)
            load_x(g + 3, 1)
            compute_idx(1)
            fire_gather(1, semB)
            return carry

        lax.fori_loop(0, (CHUNKS - 2) // 2, pipe_body, 0)

        drain_store(CHUNKS - 2, 0, semA)
        drain_store(CHUNKS - 1, 1, semB)

    return k(x_flat, tab_flat)


def kernel(x, tables):
    x_flat = jnp.transpose(x).reshape(-1)
    tab_flat = (
        tables.reshape(LEVELS, 4096, 128, FEATURES)
        .transpose(0, 1, 3, 2)
        .reshape(-1)
    )
    out = _sc_encode(x_flat, tab_flat)
    return out.reshape(N_POINTS, LEVELS * FEATURES)


# speedup vs baseline: 203.4305x; 1.4289x over previous
import functools

import jax
import jax.numpy as jnp
import numpy as np
from jax import lax
from jax.experimental import pallas as pl
from jax.experimental.pallas import tpu as pltpu
from jax.experimental.pallas import tpu_sc as plsc

LEVELS = 16
FEATURES = 2
HASHMAP_SIZE = 524288
MASK = np.int32(HASHMAP_SIZE - 1)
BASE_RES = 16
PER_LEVEL_SCALE = 1.5
N_POINTS = 1048576

P1 = np.int32(73856093)
P2 = np.int32(19349663)
P3 = np.int32(83492791)

RES = [int(BASE_RES * PER_LEVEL_SCALE**l) for l in range(LEVELS)]

NUM_CORES = 2
NUM_SUBCORES = 16
NUM_WORKERS = NUM_CORES * NUM_SUBCORES
PTS_PER_WORKER = N_POINTS // NUM_WORKERS
C = 512
CHUNKS = PTS_PER_WORKER // C
OUT_C = LEVELS * FEATURES * C


def _sc_encode(x_flat, tab_flat):
    mesh = plsc.VectorSubcoreMesh(core_axis_name="c", subcore_axis_name="s")

    @functools.partial(
        pl.kernel,
        mesh=mesh,
        out_type=jax.ShapeDtypeStruct((N_POINTS * LEVELS * FEATURES,), jnp.float32),
        compiler_params=pltpu.CompilerParams(
            needs_layout_passes=False, use_tc_tiling_on_sc=False
        ),
        scratch_types=[
            pltpu.VMEM((2, 3 * C), jnp.float32),
            pltpu.VMEM((2, OUT_C), jnp.int32),
            pltpu.VMEM((2, OUT_C), jnp.float32),
            pltpu.SemaphoreType.DMA,
            pltpu.SemaphoreType.DMA,
            pltpu.SemaphoreType.DMA,
        ],
    )
    def k(x_hbm, tab_hbm, out_hbm, xbuf, idxbuf, gbuf, semx, semA, semB):
        wid = lax.axis_index("s") * NUM_CORES + lax.axis_index("c")
        iota = lax.iota(jnp.int32, 16)
        first = wid * CHUNKS

        def load_x(g, b):
            base = (first + g) * C
            for j in range(3):
                pltpu.async_copy(
                    x_hbm.at[pl.ds(j * N_POINTS + base, C)],
                    xbuf.at[b, pl.ds(j * C, C)],
                    semx,
                )
            pltpu.make_async_copy(
                x_hbm.at[pl.ds(0, 3 * C)], xbuf.at[b], semx
            ).wait()

        def compute_idx(b):
            def vreg_body(i, carry2):
                cx = xbuf[b, pl.ds(i * 16, 16)]
                cy = xbuf[b, pl.ds(C + i * 16, 16)]
                cz = xbuf[b, pl.ds(2 * C + i * 16, 16)]
                pbase = (i >> 3) * 1024 + (i & 7) * 16 + iota
                for l in range(LEVELS):
                    res = np.float32(RES[l])
                    xi = (cx * res).astype(jnp.int32)
                    yi = (cy * res).astype(jnp.int32)
                    zi = (cz * res).astype(jnp.int32)
                    h = (xi * P1) ^ (yi * P2) ^ (zi * P3)
                    v = h & MASK
                    g0 = (
                        ((v >> 7) << 8)
                        + (v & np.int32(127))
                        + np.int32(l * 1048576)
                    )
                    for f in range(FEATURES):
                        ch = 2 * l + f
                        rb, r = ch >> 3, ch & 7
                        plsc.store_scatter(
                            idxbuf.at[b],
                            [pbase + np.int32(rb * 8 * C + r * 128)],
                            g0 + np.int32(f * 128),
                        )
                return carry2

            lax.fori_loop(0, C // 16, vreg_body, 0)

        def fire_gather(b, sem):
            pltpu.async_copy(tab_hbm.at[idxbuf.at[b]], gbuf.at[b], sem)

        def drain_store(g, b, sem):
            pltpu.make_async_copy(
                tab_hbm.at[idxbuf.at[b]], gbuf.at[b], sem
            ).wait()
            for rb in range(4):
                pltpu.sync_copy(
                    gbuf.at[b, pl.ds(rb * 8 * C, 8 * C)],
                    out_hbm.at[pl.ds(rb * 8388608 + (first + g) * C * 8, 8 * C)],
                )

        load_x(0, 0)
        compute_idx(0)
        fire_gather(0, semA)
        load_x(1, 1)
        compute_idx(1)
        fire_gather(1, semB)

        def pipe_body(jj, carry):
            g = jj * 2
            drain_store(g, 0, semA)
            load_x(g + 2, 0)
            compute_idx(0)
            fire_gather(0, semA)
            drain_store(g + 1, 1, semB)
            load_x(g + 3, 1)
            compute_idx(1)
            fire_gather(1, semB)
            return carry

        lax.fori_loop(0, (CHUNKS - 2) // 2, pipe_body, 0)

        drain_store(CHUNKS - 2, 0, semA)
        drain_store(CHUNKS - 1, 1, semB)

    return k(x_flat, tab_flat)


def kernel(x, tables):
    x_flat = jnp.transpose(x).reshape(-1)
    tab_flat = (
        tables.reshape(LEVELS, 4096, 128, FEATURES)
        .transpose(0, 1, 3, 2)
        .reshape(-1)
    )
    out = _sc_encode(x_flat, tab_flat)
    return (
        out.reshape(4, 8192, 8, 128)
        .transpose(1, 3, 0, 2)
        .reshape(N_POINTS, LEVELS * FEATURES)
    )
